# baseline (speedup 1.0000x reference)
import functools

import jax
import jax.numpy as jnp
from jax import lax
from jax.experimental import pallas as pl
from jax.experimental.pallas import tpu as pltpu
from jax.experimental.pallas import tpu_sc as plsc

_NC = 2
_NS = 16
_NW = _NC * _NS
_D = 8
_C = 128
_KB = 8


@functools.cache
def _make_sc_agg(n_nodes: int, n_chunks: int, w: int):
    lo = (n_chunks // _NW) & ~7
    extra = (n_chunks - lo * _NW) // 8
    nfull = lo // _KB
    assert lo == nfull * _KB and nfull % 3 == 0
    niter = nfull // 3
    rps = n_nodes // _NS

    mesh = plsc.VectorSubcoreMesh(core_axis_name="c", subcore_axis_name="s")

    @functools.partial(
        pl.kernel,
        mesh=mesh,
        out_type=jax.ShapeDtypeStruct((_NC, n_nodes, w), jnp.float32),
        compiler_params=pltpu.CompilerParams(use_tc_tiling_on_sc=False),
        scratch_types=[
            pltpu.VMEM_SHARED((n_nodes, w), jnp.float32),
            pltpu.VMEM_SHARED((n_nodes, w), jnp.float32),
        ] + [pltpu.VMEM((_KB * _C,), jnp.int32)] * 6
          + [pltpu.VMEM((_KB * _C, w), jnp.float32)] * 3
          + [pltpu.SemaphoreType.DMA] * 9,
    )
    def sc_agg(y_hbm, src_hbm, dst_hbm, zeros_hbm, out_hbm,
               ytab, acc, s0, d0, s1, d1, s2, d2, r0, r1, r2,
               gs0, gs1, gs2, ss0, ss1, ss2, is0, is1, is2):
        sets = [(s0, d0, r0, gs0, ss0, is0),
                (s1, d1, r1, gs1, ss1, is1),
                (s2, d2, r2, gs2, ss2, is2)]
        c = lax.axis_index("c")
        s = lax.axis_index("s")
        wid = c * _NS + s

        r0 = s * rps
        pltpu.sync_copy(y_hbm.at[pl.ds(r0, rps), :], ytab.at[pl.ds(r0, rps), :])
        pltpu.sync_copy(zeros_hbm.at[pl.ds(r0, rps), :], acc.at[pl.ds(r0, rps), :])
        plsc.subcore_barrier()

        cb = (wid * lo + 8 * jnp.minimum(wid, extra)) * _C

        eb = _KB * _C

        def fire_idx(e0, sref, dref, sem):
            return [pltpu.async_copy(src_hbm.at[pl.ds(e0, eb)], sref, sem),
                    pltpu.async_copy(dst_hbm.at[pl.ds(e0, eb)], dref, sem)]

        def fire_g(sref, rows, sem):
            return [pltpu.async_copy(ytab.at[sref], rows, sem)]

        def fire_s(dref, rows, sem):
            return [pltpu.async_copy(rows, acc.at[dref], sem, add=True)]

        def drain(descs):
            for d in descs:
                d.wait()

        def drain_s(dref, rows, sem):
            pltpu.make_async_copy(rows, acc.at[dref], sem).wait()

        def do_iter(i, first):
            e0 = cb + i * 3 * eb
            idx = []
            for k, (sk, dk, rk, gsk, ssk, isk) in enumerate(sets):
                if not first:
                    drain_s(dk, rk, ssk)
                idx.append(fire_idx(e0 + k * eb, sk, dk, isk))
            gs = []
            for k, (sk, dk, rk, gsk, ssk, isk) in enumerate(sets):
                drain(idx[k])
                gs.append(fire_g(sk, rk, gsk))
            for k, (sk, dk, rk, gsk, ssk, isk) in enumerate(sets):
                drain(gs[k])
                fire_s(dk, rk, ssk)

        do_iter(0, True)
        lax.fori_loop(1, niter, lambda i, c: (do_iter(i, False), c)[1], 0)
        for k, (sk, dk, rk, gsk, ssk, isk) in enumerate(sets):
            drain_s(dk, rk, ssk)

        if extra:
            @pl.when(wid < extra)
            def _():
                s0_, d0_, r0_, gs0_, ss0_, is0_ = sets[0]
                e0 = cb + nfull * eb
                drain(fire_idx(e0, s0_, d0_, is0_))
                drain(fire_g(s0_, r0_, gs0_))
                drain(fire_s(d0_, r0_, ss0_))

        plsc.subcore_barrier()
        pltpu.sync_copy(acc.at[pl.ds(r0, rps), :],
                        out_hbm.at[c, pl.ds(r0, rps), :])

    return sc_agg


def kernel(x, edge_index, params):
    n = x.shape[0]
    e = edge_index.shape[1]
    npad = -(-n // 128) * 128
    src = edge_index[0]
    dst = edge_index[1]
    def mean_agg(y):
        yp = jnp.pad(y, ((0, npad - n), (0, _D - y.shape[1])))
        parts = _make_sc_agg(npad, e // _C, _D)(
            yp, src, dst, jnp.zeros((npad, _D), jnp.float32))
        return parts[0, :n] + parts[1, :n]

    sa = mean_agg(x[:, :8])
    sb = mean_agg(jnp.concatenate(
        [x[:, 8:10], jnp.ones((n, 1), jnp.float32)], axis=1))
    cnt = jnp.clip(sb[:, 2:3], 1.0, None)
    agg = jnp.concatenate([sa, sb[:, :2]], axis=1) / cnt

    h = x
    for l, p in enumerate(params):
        h = agg @ p["Wl"] + p["bl"] + h @ p["Wr"]
        if l < len(params) - 1:
            h = jax.nn.relu(h)
            agg = mean_agg(h)[:, :h.shape[1]] / cnt
    return h


# speedup vs baseline: 1.0041x; 1.0041x over previous
import functools

import jax
import jax.numpy as jnp
from jax import lax
from jax.experimental import pallas as pl
from jax.experimental.pallas import tpu as pltpu
from jax.experimental.pallas import tpu_sc as plsc

_NC = 2
_NS = 16
_NW = _NC * _NS
_D = 8
_C = 128
_KB = 8


@functools.cache
def _make_sc_agg(n_nodes: int, n_chunks: int, w: int):
    lo = (n_chunks // _NW) & ~7
    extra = (n_chunks - lo * _NW) // 8
    nfull = lo // _KB
    assert lo == nfull * _KB and nfull % 3 == 0
    niter = nfull // 3
    rps = n_nodes // _NS

    mesh = plsc.VectorSubcoreMesh(core_axis_name="c", subcore_axis_name="s")

    @functools.partial(
        pl.kernel,
        mesh=mesh,
        out_type=jax.ShapeDtypeStruct((_NC, n_nodes, w), jnp.float32),
        compiler_params=pltpu.CompilerParams(use_tc_tiling_on_sc=False),
        scratch_types=[
            pltpu.VMEM_SHARED((n_nodes, w), jnp.float32),
            pltpu.VMEM_SHARED((n_nodes, w), jnp.float32),
        ] + [pltpu.VMEM((_KB * _C,), jnp.int32)] * 6
          + [pltpu.VMEM((_KB * _C, w), jnp.float32)] * 3
          + [pltpu.SemaphoreType.DMA] * 9,
    )
    def sc_agg(y_hbm, src_hbm, dst_hbm, zeros_hbm, out_hbm,
               ytab, acc, s0, d0, s1, d1, s2, d2, r0, r1, r2,
               gs0, gs1, gs2, ss0, ss1, ss2, is0, is1, is2):
        sets = [(s0, d0, r0, gs0, ss0, is0),
                (s1, d1, r1, gs1, ss1, is1),
                (s2, d2, r2, gs2, ss2, is2)]
        c = lax.axis_index("c")
        s = lax.axis_index("s")
        wid = c * _NS + s

        cb = (wid * lo + 8 * jnp.minimum(wid, extra)) * _C

        eb = _KB * _C

        def fire_idx(e0, sref, dref, sem):
            return [pltpu.async_copy(src_hbm.at[pl.ds(e0, eb)], sref, sem),
                    pltpu.async_copy(dst_hbm.at[pl.ds(e0, eb)], dref, sem)]

        def fire_g(sref, rows, sem):
            return [pltpu.async_copy(ytab.at[sref], rows, sem)]

        def fire_s(dref, rows, sem):
            return [pltpu.async_copy(rows, acc.at[dref], sem, add=True)]

        def drain(descs):
            for d in descs:
                d.wait()

        def drain_s(dref, rows, sem):
            pltpu.make_async_copy(rows, acc.at[dref], sem).wait()

        def finish_iter(idx):
            gs = []
            for k, (sk, dk, rk, gsk, ssk, isk) in enumerate(sets):
                drain(idx[k])
                gs.append(fire_g(sk, rk, gsk))
            for k, (sk, dk, rk, gsk, ssk, isk) in enumerate(sets):
                drain(gs[k])
                fire_s(dk, rk, ssk)

        def do_iter(i, first):
            e0 = cb + i * 3 * eb
            idx = []
            for k, (sk, dk, rk, gsk, ssk, isk) in enumerate(sets):
                if not first:
                    drain_s(dk, rk, ssk)
                idx.append(fire_idx(e0 + k * eb, sk, dk, isk))
            finish_iter(idx)

        idx0 = [fire_idx(cb + k * eb, sets[k][0], sets[k][1], sets[k][5])
                for k in range(3)]
        r0 = s * rps
        pltpu.sync_copy(y_hbm.at[pl.ds(r0, rps), :], ytab.at[pl.ds(r0, rps), :])
        pltpu.sync_copy(zeros_hbm.at[pl.ds(r0, rps), :], acc.at[pl.ds(r0, rps), :])
        plsc.subcore_barrier()

        finish_iter(idx0)
        lax.fori_loop(1, niter, lambda i, c: (do_iter(i, False), c)[1], 0)
        for k, (sk, dk, rk, gsk, ssk, isk) in enumerate(sets):
            drain_s(dk, rk, ssk)

        if extra:
            @pl.when(wid < extra)
            def _():
                s0_, d0_, r0_, gs0_, ss0_, is0_ = sets[0]
                e0 = cb + nfull * eb
                drain(fire_idx(e0, s0_, d0_, is0_))
                drain(fire_g(s0_, r0_, gs0_))
                drain(fire_s(d0_, r0_, ss0_))

        plsc.subcore_barrier()
        pltpu.sync_copy(acc.at[pl.ds(r0, rps), :],
                        out_hbm.at[c, pl.ds(r0, rps), :])

    return sc_agg


def kernel(x, edge_index, params):
    n = x.shape[0]
    e = edge_index.shape[1]
    npad = -(-n // 128) * 128
    src = edge_index[0]
    dst = edge_index[1]
    def mean_agg(y):
        yp = jnp.pad(y, ((0, npad - n), (0, _D - y.shape[1])))
        parts = _make_sc_agg(npad, e // _C, _D)(
            yp, src, dst, jnp.zeros((npad, _D), jnp.float32))
        return parts[0, :n] + parts[1, :n]

    sa = mean_agg(x[:, :8])
    sb = mean_agg(jnp.concatenate(
        [x[:, 8:10], jnp.ones((n, 1), jnp.float32)], axis=1))
    cnt = jnp.clip(sb[:, 2:3], 1.0, None)
    agg = jnp.concatenate([sa, sb[:, :2]], axis=1) / cnt

    h = x
    for l, p in enumerate(params):
        h = agg @ p["Wl"] + p["bl"] + h @ p["Wr"]
        if l < len(params) - 1:
            h = jax.nn.relu(h)
            agg = mean_agg(h)[:, :h.shape[1]] / cnt
    return h
